# baseline (speedup 1.0000x reference)
import functools

import jax
import jax.numpy as jnp
from jax.experimental import pallas as pl
from jax.experimental.pallas import tpu as pltpu


def _gcn_body(x_ref, adj_ref, w1_ref, b1_ref, w2_ref, b2_ref,
              feat_ref, out_ref, support_ref):
    i = pl.program_id(0)

    @pl.when(i == 0)
    def _compute_support():
        support_ref[...] = (
            jnp.dot(x_ref[...].astype(jnp.bfloat16),
                    w1_ref[...].astype(jnp.bfloat16),
                    preferred_element_type=jnp.float32)
            + b1_ref[...]
        ).astype(jnp.bfloat16)

    h = jnp.dot(adj_ref[...].astype(jnp.bfloat16), support_ref[...],
                preferred_element_type=jnp.float32)
    feat = jnp.maximum(h, 0.0)
    feat_ref[...] = feat
    out_ref[...] = jax.nn.sigmoid(
        jnp.dot(feat.astype(jnp.bfloat16), w2_ref[...].astype(jnp.bfloat16),
                preferred_element_type=jnp.float32)
        + b2_ref[...]
    )


@functools.partial(jax.jit, static_argnames=("block_n",))
def _gcn_fused(x, adj, W1, b1, W2, b2, block_n=512):
    n, f = x.shape
    h_dim = W1.shape[1]
    c = W2.shape[1]
    b1r = b1.reshape(1, h_dim)
    b2r = b2.reshape(1, c)
    feature, out = pl.pallas_call(
        _gcn_body,
        grid=(n // block_n,),
        in_specs=[
            pl.BlockSpec((n, f), lambda i: (0, 0)),
            pl.BlockSpec((block_n, n), lambda i: (i, 0)),
            pl.BlockSpec((f, h_dim), lambda i: (0, 0)),
            pl.BlockSpec((1, h_dim), lambda i: (0, 0)),
            pl.BlockSpec((h_dim, c), lambda i: (0, 0)),
            pl.BlockSpec((1, c), lambda i: (0, 0)),
        ],
        out_specs=[
            pl.BlockSpec((block_n, h_dim), lambda i: (i, 0)),
            pl.BlockSpec((block_n, c), lambda i: (i, 0)),
        ],
        out_shape=[
            jax.ShapeDtypeStruct((n, h_dim), jnp.float32),
            jax.ShapeDtypeStruct((n, c), jnp.float32),
        ],
        scratch_shapes=[pltpu.VMEM((n, h_dim), jnp.bfloat16)],
    )(x, adj, W1, b1r, W2, b2r)
    return feature, out


def kernel(x, adj, W1, b1, W2, b2):
    return _gcn_fused(x, adj, W1, b1, W2, b2, block_n=256)


# speedup vs baseline: 1.1388x; 1.1388x over previous
import functools

import jax
import jax.numpy as jnp
from jax.experimental import pallas as pl
from jax.experimental.pallas import tpu as pltpu


def _gcn_body(x_ref, adj_ref, w1_ref, b1_ref, w2_ref, b2_ref,
              feat_ref, out_ref, support_ref):
    i = pl.program_id(0)

    @pl.when(i == 0)
    def _compute_support():
        support_ref[...] = (
            jnp.dot(x_ref[...].astype(jnp.bfloat16),
                    w1_ref[...].astype(jnp.bfloat16),
                    preferred_element_type=jnp.float32)
            + b1_ref[...]
        ).astype(jnp.bfloat16)

    h = jnp.dot(adj_ref[...].astype(jnp.bfloat16), support_ref[...],
                preferred_element_type=jnp.float32)
    feat = jnp.maximum(h, 0.0)
    feat_ref[...] = feat
    out_ref[...] = jax.nn.sigmoid(
        jnp.dot(feat.astype(jnp.bfloat16), w2_ref[...].astype(jnp.bfloat16),
                preferred_element_type=jnp.float32)
        + b2_ref[...]
    )


@functools.partial(jax.jit, static_argnames=("block_n",))
def _gcn_fused(x, adj, W1, b1, W2, b2, block_n=512):
    n, f = x.shape
    h_dim = W1.shape[1]
    c = W2.shape[1]
    b1r = b1.reshape(1, h_dim)
    b2r = b2.reshape(1, c)
    feature, out = pl.pallas_call(
        _gcn_body,
        grid=(n // block_n,),
        in_specs=[
            pl.BlockSpec((n, f), lambda i: (0, 0)),
            pl.BlockSpec((block_n, n), lambda i: (i, 0)),
            pl.BlockSpec((f, h_dim), lambda i: (0, 0)),
            pl.BlockSpec((1, h_dim), lambda i: (0, 0)),
            pl.BlockSpec((h_dim, c), lambda i: (0, 0)),
            pl.BlockSpec((1, c), lambda i: (0, 0)),
        ],
        out_specs=[
            pl.BlockSpec((block_n, h_dim), lambda i: (i, 0)),
            pl.BlockSpec((block_n, c), lambda i: (i, 0)),
        ],
        out_shape=[
            jax.ShapeDtypeStruct((n, h_dim), jnp.float32),
            jax.ShapeDtypeStruct((n, c), jnp.float32),
        ],
        scratch_shapes=[pltpu.VMEM((n, h_dim), jnp.bfloat16)],
    )(x, adj, W1, b1r, W2, b2r)
    return feature, out


def kernel(x, adj, W1, b1, W2, b2):
    return _gcn_fused(x, adj, W1, b1, W2, b2, block_n=1024)
